# baseline (speedup 1.0000x reference)
import functools

import jax
import jax.numpy as jnp
from jax.experimental import pallas as pl
from jax.experimental.pallas import tpu as pltpu

EMBED_DIM = 2048
ROUTER_HIDDEN = 64
NUM_EXPERTS = 16
TOP_K = 2
BLOCK_M = 512


def _half(xb, g, b, w1, w2):
    mean = jnp.mean(xb, axis=1, keepdims=True)
    xc = xb - mean
    var = jnp.mean(xc * xc, axis=1, keepdims=True)
    xn = xc * jax.lax.rsqrt(var + 1e-5) * g + b
    h = jnp.dot(xn, w1, preferred_element_type=jnp.float32)
    h = 0.5 * h * (1.0 + jax.lax.erf(h * 0.70710678118654752))
    logits = jnp.dot(h, w2, preferred_element_type=jnp.float32) * 0.5
    m = jnp.max(logits, axis=1, keepdims=True)
    e = jnp.exp(logits - m)
    s = jnp.sum(e, axis=1, keepdims=True)
    p = e / s
    lane = jax.lax.broadcasted_iota(jnp.int32, p.shape, 1)
    m1 = jnp.max(p, axis=1, keepdims=True)
    i1 = jnp.min(jnp.where(p == m1, lane, NUM_EXPERTS), axis=1, keepdims=True)
    p2 = jnp.where(lane == i1, -1.0, p)
    m2 = jnp.max(p2, axis=1, keepdims=True)
    i2 = jnp.min(jnp.where(p2 == m2, lane, NUM_EXPERTS), axis=1, keepdims=True)
    idx = jnp.concatenate([i1, i2], axis=1)
    hit1 = (lane == i1).astype(jnp.float32)
    hit2 = (lane == i2).astype(jnp.float32)
    cnt = jnp.sum(hit1 + hit2, axis=0, keepdims=True)
    psum = jnp.sum(p, axis=0, keepdims=True)
    return p, idx, cnt, psum


def _router_block(xa_ref, xb_ref, g_ref, b_ref, w1_ref, w2_ref,
                  probs_ref, idx_ref, aux_ref, acc_ref, *, nsteps, n_tokens):
    step = pl.program_id(0)

    @pl.when(step == 0)
    def _init():
        acc_ref[...] = jnp.zeros_like(acc_ref)

    g = g_ref[...]
    b = b_ref[...]
    w1 = w1_ref[...]
    w2 = w2_ref[...]
    pa, ia, cnta, psa = _half(xa_ref[...], g, b, w1, w2)
    pb, ib, cntb, psb = _half(xb_ref[...], g, b, w1, w2)
    probs_ref[0:BLOCK_M, :] = pa
    probs_ref[BLOCK_M:2 * BLOCK_M, :] = pb
    idx_ref[0:BLOCK_M, :] = ia
    idx_ref[BLOCK_M:2 * BLOCK_M, :] = ib
    acc_ref[0:1, :] = acc_ref[0:1, :] + cnta + cntb
    acc_ref[1:2, :] = acc_ref[1:2, :] + psa + psb

    @pl.when(step == nsteps - 1)
    def _finalize():
        inv = 1.0 / (jnp.float32(n_tokens) * TOP_K * jnp.float32(n_tokens))
        aux = NUM_EXPERTS * jnp.sum(acc_ref[0:1, :] * acc_ref[1:2, :],
                                    keepdims=True) * inv
        aux_ref[...] = aux


def kernel(x, ln_g, ln_b, W1, W2):
    B, L, D = x.shape
    n = B * L
    x2 = x.reshape(n, D)
    g2 = ln_g.reshape(1, D)
    b2 = ln_b.reshape(1, D)
    w1t = W1.T
    w2t = W2.T
    nsteps = n // (2 * BLOCK_M)

    probs, idx, aux = pl.pallas_call(
        functools.partial(_router_block, nsteps=nsteps, n_tokens=n),
        grid=(nsteps,),
        in_specs=[
            pl.BlockSpec((BLOCK_M, D), lambda i: (2 * i, 0)),
            pl.BlockSpec((BLOCK_M, D), lambda i: (2 * i + 1, 0)),
            pl.BlockSpec((1, D), lambda i: (0, 0)),
            pl.BlockSpec((1, D), lambda i: (0, 0)),
            pl.BlockSpec((D, ROUTER_HIDDEN), lambda i: (0, 0)),
            pl.BlockSpec((ROUTER_HIDDEN, NUM_EXPERTS), lambda i: (0, 0)),
        ],
        out_specs=[
            pl.BlockSpec((2 * BLOCK_M, NUM_EXPERTS), lambda i: (i, 0)),
            pl.BlockSpec((2 * BLOCK_M, TOP_K), lambda i: (i, 0)),
            pl.BlockSpec((1, 1), lambda i: (0, 0)),
        ],
        out_shape=[
            jax.ShapeDtypeStruct((n, NUM_EXPERTS), jnp.float32),
            jax.ShapeDtypeStruct((n, TOP_K), jnp.int32),
            jax.ShapeDtypeStruct((1, 1), jnp.float32),
        ],
        scratch_shapes=[pltpu.VMEM((2, NUM_EXPERTS), jnp.float32)],
    )(x2, x2, g2, b2, w1t, w2t)

    probs = probs.reshape(B, L, NUM_EXPERTS)
    idx = idx.reshape(B, L, TOP_K)
    aux_loss = aux[0, 0]
    return (probs, idx, aux_loss, probs)


# speedup vs baseline: 1.0852x; 1.0852x over previous
import functools

import jax
import jax.numpy as jnp
from jax.experimental import pallas as pl
from jax.experimental.pallas import tpu as pltpu

EMBED_DIM = 2048
ROUTER_HIDDEN = 64
NUM_EXPERTS = 16
TOP_K = 2
BLOCK_M = 1024


def _router_block(x_ref, wg_ref, bb_ref, w2_ref,
                  probs_ref, idx_ref, aux_ref, acc_ref, *, nsteps, n_tokens):
    step = pl.program_id(0)

    @pl.when(step == 0)
    def _init():
        acc_ref[...] = jnp.zeros_like(acc_ref)

    xb = x_ref[...]
    mean = jnp.mean(xb, axis=1, keepdims=True)
    xc = xb - mean
    var = jnp.mean(xc * xc, axis=1, keepdims=True)
    rstd = jax.lax.rsqrt(var + 1e-5)
    y = jnp.dot(xc, wg_ref[...], preferred_element_type=jnp.float32)
    h = y * rstd + bb_ref[...]
    h = 0.5 * h * (1.0 + jax.lax.erf(h * 0.70710678118654752))
    logits = jnp.dot(h, w2_ref[...], preferred_element_type=jnp.float32) * 0.5
    m = jnp.max(logits, axis=1, keepdims=True)
    e = jnp.exp(logits - m)
    s = jnp.sum(e, axis=1, keepdims=True)
    p = e / s
    probs_ref[...] = p

    lane = jax.lax.broadcasted_iota(jnp.int32, p.shape, 1)
    m1 = jnp.max(p, axis=1, keepdims=True)
    i1 = jnp.min(jnp.where(p == m1, lane, NUM_EXPERTS), axis=1, keepdims=True)
    p2 = jnp.where(lane == i1, -1.0, p)
    m2 = jnp.max(p2, axis=1, keepdims=True)
    i2 = jnp.min(jnp.where(p2 == m2, lane, NUM_EXPERTS), axis=1, keepdims=True)
    idx_ref[...] = jnp.concatenate([i1, i2], axis=1)

    hit1 = (lane == i1).astype(jnp.float32)
    hit2 = (lane == i2).astype(jnp.float32)
    cnt = jnp.sum(hit1 + hit2, axis=0, keepdims=True)
    psum = jnp.sum(p, axis=0, keepdims=True)
    acc_ref[0:1, :] = acc_ref[0:1, :] + cnt
    acc_ref[1:2, :] = acc_ref[1:2, :] + psum

    @pl.when(step == nsteps - 1)
    def _finalize():
        inv = 1.0 / (jnp.float32(n_tokens) * TOP_K * jnp.float32(n_tokens))
        aux = NUM_EXPERTS * jnp.sum(acc_ref[0:1, :] * acc_ref[1:2, :],
                                    keepdims=True) * inv
        aux_ref[...] = aux


def kernel(x, ln_g, ln_b, W1, W2):
    B, L, D = x.shape
    n = B * L
    x2 = x.reshape(n, D)
    wg = (W1 * ln_g[None, :]).T
    bb = (W1 @ ln_b).reshape(1, ROUTER_HIDDEN)
    w2t = W2.T
    nsteps = n // BLOCK_M

    probs, idx, aux = pl.pallas_call(
        functools.partial(_router_block, nsteps=nsteps, n_tokens=n),
        grid=(nsteps,),
        in_specs=[
            pl.BlockSpec((BLOCK_M, D), lambda i: (i, 0)),
            pl.BlockSpec((D, ROUTER_HIDDEN), lambda i: (0, 0)),
            pl.BlockSpec((1, ROUTER_HIDDEN), lambda i: (0, 0)),
            pl.BlockSpec((ROUTER_HIDDEN, NUM_EXPERTS), lambda i: (0, 0)),
        ],
        out_specs=[
            pl.BlockSpec((BLOCK_M, NUM_EXPERTS), lambda i: (i, 0)),
            pl.BlockSpec((BLOCK_M, TOP_K), lambda i: (i, 0)),
            pl.BlockSpec((1, 1), lambda i: (0, 0)),
        ],
        out_shape=[
            jax.ShapeDtypeStruct((n, NUM_EXPERTS), jnp.float32),
            jax.ShapeDtypeStruct((n, TOP_K), jnp.int32),
            jax.ShapeDtypeStruct((1, 1), jnp.float32),
        ],
        scratch_shapes=[pltpu.VMEM((2, NUM_EXPERTS), jnp.float32)],
    )(x2, wg, bb, w2t)

    probs = probs.reshape(B, L, NUM_EXPERTS)
    idx = idx.reshape(B, L, TOP_K)
    aux_loss = aux[0, 0]
    return (probs, idx, aux_loss, probs)


# speedup vs baseline: 1.0990x; 1.0127x over previous
import functools

import jax
import jax.numpy as jnp
from jax.experimental import pallas as pl
from jax.experimental.pallas import tpu as pltpu

EMBED_DIM = 2048
ROUTER_HIDDEN = 64
NUM_EXPERTS = 16
TOP_K = 2
BLOCK_M = 2048


def _router_block(x_ref, g_ref, b_ref, w1_ref, w2_ref,
                  probs_ref, idx_ref, aux_ref, acc_ref, *, nsteps, n_tokens):
    step = pl.program_id(0)

    @pl.when(step == 0)
    def _init():
        acc_ref[...] = jnp.zeros_like(acc_ref)

    xb = x_ref[...]
    mean = jnp.mean(xb, axis=1, keepdims=True)
    xc = xb - mean
    var = jnp.mean(xc * xc, axis=1, keepdims=True)
    xn = xc * jax.lax.rsqrt(var + 1e-5) * g_ref[...] + b_ref[...]
    h = jnp.dot(xn, w1_ref[...], preferred_element_type=jnp.float32)
    h = 0.5 * h * (1.0 + jax.lax.erf(h * 0.70710678118654752))
    logits = jnp.dot(h, w2_ref[...], preferred_element_type=jnp.float32) * 0.5
    m = jnp.max(logits, axis=1, keepdims=True)
    e = jnp.exp(logits - m)
    s = jnp.sum(e, axis=1, keepdims=True)
    p = e / s
    probs_ref[...] = p

    lane = jax.lax.broadcasted_iota(jnp.int32, p.shape, 1)
    m1 = jnp.max(p, axis=1, keepdims=True)
    i1 = jnp.min(jnp.where(p == m1, lane, NUM_EXPERTS), axis=1, keepdims=True)
    p2 = jnp.where(lane == i1, -1.0, p)
    m2 = jnp.max(p2, axis=1, keepdims=True)
    i2 = jnp.min(jnp.where(p2 == m2, lane, NUM_EXPERTS), axis=1, keepdims=True)
    idx_ref[...] = jnp.concatenate([i1, i2], axis=1)

    hit1 = (lane == i1).astype(jnp.float32)
    hit2 = (lane == i2).astype(jnp.float32)
    cnt = jnp.sum(hit1 + hit2, axis=0, keepdims=True)
    psum = jnp.sum(p, axis=0, keepdims=True)
    acc_ref[0:1, :] = acc_ref[0:1, :] + cnt
    acc_ref[1:2, :] = acc_ref[1:2, :] + psum

    @pl.when(step == nsteps - 1)
    def _finalize():
        inv = 1.0 / (jnp.float32(n_tokens) * TOP_K * jnp.float32(n_tokens))
        aux = NUM_EXPERTS * jnp.sum(acc_ref[0:1, :] * acc_ref[1:2, :],
                                    keepdims=True) * inv
        aux_ref[...] = aux


def kernel(x, ln_g, ln_b, W1, W2):
    B, L, D = x.shape
    n = B * L
    x2 = x.reshape(n, D)
    g2 = ln_g.reshape(1, D)
    b2 = ln_b.reshape(1, D)
    w1t = W1.T
    w2t = W2.T
    nsteps = n // BLOCK_M

    probs, idx, aux = pl.pallas_call(
        functools.partial(_router_block, nsteps=nsteps, n_tokens=n),
        grid=(nsteps,),
        in_specs=[
            pl.BlockSpec((BLOCK_M, D), lambda i: (i, 0)),
            pl.BlockSpec((1, D), lambda i: (0, 0)),
            pl.BlockSpec((1, D), lambda i: (0, 0)),
            pl.BlockSpec((D, ROUTER_HIDDEN), lambda i: (0, 0)),
            pl.BlockSpec((ROUTER_HIDDEN, NUM_EXPERTS), lambda i: (0, 0)),
        ],
        out_specs=[
            pl.BlockSpec((BLOCK_M, NUM_EXPERTS), lambda i: (i, 0)),
            pl.BlockSpec((BLOCK_M, TOP_K), lambda i: (i, 0)),
            pl.BlockSpec((1, 1), lambda i: (0, 0)),
        ],
        out_shape=[
            jax.ShapeDtypeStruct((n, NUM_EXPERTS), jnp.float32),
            jax.ShapeDtypeStruct((n, TOP_K), jnp.int32),
            jax.ShapeDtypeStruct((1, 1), jnp.float32),
        ],
        scratch_shapes=[pltpu.VMEM((2, NUM_EXPERTS), jnp.float32)],
    )(x2, g2, b2, w1t, w2t)

    probs = probs.reshape(B, L, NUM_EXPERTS)
    idx = idx.reshape(B, L, TOP_K)
    aux_loss = aux[0, 0]
    return (probs, idx, aux_loss, probs)
